# Initial kernel scaffold; baseline (speedup 1.0000x reference)
import jax
import jax.numpy as jnp
from jax.experimental import pallas as pl


def kernel(x, edge_index, W1, a_src1, a_dst1, b1, W2, a_src2, a_dst2, b2):
    raise NotImplementedError("write your pallas kernel here")


# speedup vs baseline: 35.4035x; 35.4035x over previous
import functools
import jax
import jax.numpy as jnp
from jax import lax
from jax.experimental import pallas as pl
from jax.experimental.pallas import tpu as pltpu
from jax.experimental.pallas import tpu_sc as plsc

N = 10000
E = 320000
NFEAT = 128
NHID = 8
NHEADS = 8
NCLASS = 40
NEG_SLOPE = 0.2

NPAD = 10112
ROWS_PER_TILE = NPAD // 16
NC = 2
NS = 16
NW = NC * NS
EPW = E // NW
CH = 80
NCH = EPW // CH
IDXROWS = E // CH

D1 = 80
D2 = 48

_f32 = jnp.float32
_i32 = jnp.int32


def _tc_stage_a(x_ref, w_ref, t1_ref, ad_ref):
    r = jnp.dot(x_ref[...], w_ref[...], preferred_element_type=_f32)
    t1_ref[...] = r[:, :D1]
    ad_ref[...] = r[:, D1:D1 + 8]


def _tc_stage_c(part_ref, k8_ref, b1_ref, w2_ref, t2_ref, ad_ref):
    p = part_ref[0] + part_ref[1]
    denom = p[:, 64:72]
    rec = jnp.where(denom > 0.0, 1.0 / denom, 0.0)
    rep = jnp.dot(rec, k8_ref[...], preferred_element_type=_f32)
    hn = p[:, :64] * rep + b1_ref[...]
    t = jnp.where(hn > 0.0, hn, jnp.exp(hn) - 1.0)
    r2 = jnp.dot(t, w2_ref[...], preferred_element_type=_f32)
    t2_ref[...] = r2[:, :D2]
    ad_ref[...] = r2[:, D2:D2 + 8]


def _tc_stage_d(part_ref, b2_ref, out_ref):
    p = part_ref[0] + part_ref[1]
    denom = p[:, 40:41]
    rec = jnp.where(denom > 0.0, 1.0 / denom, 0.0)
    z = p[:, :NCLASS] * rec + b2_ref[...]
    m = jnp.max(z, axis=1, keepdims=True)
    lse = jnp.log(jnp.sum(jnp.exp(z - m), axis=1, keepdims=True)) + m
    out_ref[...] = z - lse


def _zero_buf(buf, rows, width):
    zeros16 = jnp.zeros((16,), _f32)
    ngroups = width // 16

    def body(g, _):
        r = g // ngroups
        c = 16 * (g - r * ngroups)
        buf[r, pl.ds(c, 16)] = zeros16
        return 0

    lax.fori_loop(0, rows * ngroups, body, 0)


def _dump_acc(acc, dump, part, cid, sid):
    r0 = sid * ROWS_PER_TILE
    pltpu.sync_copy(acc.at[pl.ds(r0, ROWS_PER_TILE)], dump)
    pltpu.sync_copy(dump, part.at[cid, pl.ds(r0, ROWS_PER_TILE)])


def _sc_layer1(t1_hbm, ad_hbm, src_hbm, dst_hbm, part_hbm,
               srci_v, dsti_v, buf_v, adbuf_v, dump_v, acc_sh, gsem):
    cid = lax.axis_index("c")
    sid = lax.axis_index("s")
    wid = sid * NC + cid
    iota = lax.iota(_i32, 16)
    hi8 = iota >> 3
    lo8 = iota & 7

    _zero_buf(dump_v, ROWS_PER_TILE, D1)
    pltpu.sync_copy(dump_v, acc_sh.at[pl.ds(sid * ROWS_PER_TILE,
                                            ROWS_PER_TILE)])
    plsc.subcore_barrier()

    def chunk_body(c, _):
        row = wid * NCH + c
        pltpu.sync_copy(src_hbm.at[row], srci_v)
        pltpu.sync_copy(dst_hbm.at[row], dsti_v)
        pltpu.async_copy(t1_hbm.at[srci_v.at[0]], buf_v, gsem).wait()
        pltpu.async_copy(ad_hbm.at[dsti_v.at[0]], adbuf_v, gsem).wait()

        def att_body(i, _):
            r = 2 * i + hi8
            a_s = plsc.load_gather(buf_v, [r, 64 + lo8])
            a_d = plsc.load_gather(adbuf_v, [r, lo8])
            e = a_s + a_d
            e = jnp.where(e >= 0.0, e, NEG_SLOPE * e)
            plsc.store_scatter(buf_v, [r, 64 + lo8], jnp.exp(e))
            return 0

        lax.fori_loop(0, CH // 2, att_body, 0)

        def msg_body(g, _):
            e_row = g >> 2
            jj = g & 3
            col0 = 16 * jj
            mult = plsc.load_gather(
                buf_v, [jnp.full((16,), e_row, _i32), 64 + 2 * jj + hi8])
            buf_v[e_row, pl.ds(col0, 16)] = buf_v[e_row, pl.ds(col0, 16)] * mult
            return 0

        lax.fori_loop(0, CH * 4, msg_body, 0)

        pltpu.sync_copy(buf_v, acc_sh.at[dsti_v.at[0]], add=True)
        return 0

    lax.fori_loop(0, NCH, chunk_body, 0)

    plsc.subcore_barrier()
    _dump_acc(acc_sh, dump_v, part_hbm, cid, sid)


def _sc_layer2(t2_hbm, ad_hbm, src_hbm, dst_hbm, part_hbm,
               srci_v, dsti_v, buf_v, adbuf_v, dump_v, acc_sh, gsem):
    cid = lax.axis_index("c")
    sid = lax.axis_index("s")
    wid = sid * NC + cid
    iota = lax.iota(_i32, 16)

    _zero_buf(dump_v, ROWS_PER_TILE, D2)
    pltpu.sync_copy(dump_v, acc_sh.at[pl.ds(sid * ROWS_PER_TILE,
                                            ROWS_PER_TILE)])
    plsc.subcore_barrier()

    def chunk_body(c, _):
        row = wid * NCH + c
        pltpu.sync_copy(src_hbm.at[row], srci_v)
        pltpu.sync_copy(dst_hbm.at[row], dsti_v)
        pltpu.async_copy(t2_hbm.at[srci_v.at[0]], buf_v, gsem).wait()
        pltpu.async_copy(ad_hbm.at[dsti_v.at[0]], adbuf_v, gsem).wait()

        def att_body(i, _):
            e_idx = 16 * i + iota
            a_s = plsc.load_gather(buf_v, [e_idx, jnp.full((16,), 41, _i32)])
            a_d = plsc.load_gather(adbuf_v, [e_idx, jnp.zeros((16,), _i32)])
            e = a_s + a_d
            e = jnp.where(e >= 0.0, e, NEG_SLOPE * e)
            plsc.store_scatter(buf_v, [e_idx, jnp.full((16,), 40, _i32)],
                               jnp.exp(e))
            return 0

        lax.fori_loop(0, CH // 16, att_body, 0)

        def msg_body(g, _):
            e_row = g // 3
            jj = g - 3 * e_row
            col0 = 16 * jj
            ex = plsc.load_gather(
                buf_v, [jnp.full((16,), e_row, _i32),
                        jnp.full((16,), 40, _i32)])
            mult = jnp.where(col0 + iota < NCLASS, ex, 1.0)
            buf_v[e_row, pl.ds(col0, 16)] = buf_v[e_row, pl.ds(col0, 16)] * mult
            return 0

        lax.fori_loop(0, CH * 3, msg_body, 0)

        pltpu.sync_copy(buf_v, acc_sh.at[dsti_v.at[0]], add=True)
        return 0

    lax.fori_loop(0, NCH, chunk_body, 0)

    plsc.subcore_barrier()
    _dump_acc(acc_sh, dump_v, part_hbm, cid, sid)


def _edge_pass(layer_body, width, table, adtable, srcg, dstg):
    mesh = plsc.VectorSubcoreMesh(core_axis_name="c", subcore_axis_name="s",
                                  num_cores=NC, num_subcores=NS)
    f = pl.kernel(
        layer_body,
        out_type=jax.ShapeDtypeStruct((NC, NPAD, width), _f32),
        mesh=mesh,
        scratch_types=[
            pltpu.VMEM((1, CH), _i32),
            pltpu.VMEM((1, CH), _i32),
            pltpu.VMEM((CH, width), _f32),
            pltpu.VMEM((CH, 8), _f32),
            pltpu.VMEM((ROWS_PER_TILE, width), _f32),
            pltpu.VMEM_SHARED((NPAD, width), _f32),
            pltpu.SemaphoreType.DMA,
        ],
        compiler_params=pltpu.CompilerParams(needs_layout_passes=False,
                                             use_tc_tiling_on_sc=False),
    )
    return f(table, adtable, srcg, dstg)


@jax.jit
def kernel(x, edge_index, W1, a_src1, a_dst1, b1, W2, a_src2, a_dst2, b2):
    eye8 = jnp.eye(NHEADS, dtype=_f32)
    s_src = (eye8[:, None, :] * a_src1[:, :, None]).reshape(64, 8)
    s_dst = (eye8[:, None, :] * a_dst1[:, :, None]).reshape(64, 8)
    wfull = jnp.concatenate(
        [W1, W1 @ s_src, jnp.zeros((NFEAT, 8), _f32), W1 @ s_dst], axis=1)
    v_s2 = (W2 @ a_src2[0][:, None])
    v_d2 = (W2 @ a_dst2[0][:, None])
    z1 = jnp.zeros((64, 1), _f32)
    w2full = jnp.concatenate(
        [W2, z1, v_s2, jnp.tile(z1, (1, 6)), v_d2, jnp.tile(z1, (1, 7))],
        axis=1)
    k8 = (eye8[:, :, None] * jnp.ones((1, 1, 8), _f32)).reshape(8, 64)
    b1row = b1.reshape(1, 64)
    b2row = b2.reshape(1, NCLASS)

    xp = jnp.zeros((NPAD, NFEAT), _f32).at[:N].set(x)
    srcg = edge_index[0].reshape(IDXROWS, 1, CH)
    dstg = edge_index[1].reshape(IDXROWS, 1, CH)

    nblk = NPAD // 128

    t1, ad1 = pl.pallas_call(
        _tc_stage_a,
        grid=(nblk,),
        in_specs=[
            pl.BlockSpec((128, NFEAT), lambda i: (i, 0)),
            pl.BlockSpec((NFEAT, 88), lambda i: (0, 0)),
        ],
        out_specs=[
            pl.BlockSpec((128, D1), lambda i: (i, 0)),
            pl.BlockSpec((128, 8), lambda i: (i, 0)),
        ],
        out_shape=[
            jax.ShapeDtypeStruct((NPAD, D1), _f32),
            jax.ShapeDtypeStruct((NPAD, 8), _f32),
        ],
    )(xp, wfull)

    part1 = _edge_pass(_sc_layer1, D1, t1, ad1, srcg, dstg)

    t2, ad2 = pl.pallas_call(
        _tc_stage_c,
        grid=(nblk,),
        in_specs=[
            pl.BlockSpec((NC, 128, D1), lambda i: (0, i, 0)),
            pl.BlockSpec((8, 64), lambda i: (0, 0)),
            pl.BlockSpec((1, 64), lambda i: (0, 0)),
            pl.BlockSpec((64, 56), lambda i: (0, 0)),
        ],
        out_specs=[
            pl.BlockSpec((128, D2), lambda i: (i, 0)),
            pl.BlockSpec((128, 8), lambda i: (i, 0)),
        ],
        out_shape=[
            jax.ShapeDtypeStruct((NPAD, D2), _f32),
            jax.ShapeDtypeStruct((NPAD, 8), _f32),
        ],
    )(part1, k8, b1row, w2full)

    part2 = _edge_pass(_sc_layer2, D2, t2, ad2, srcg, dstg)

    out = pl.pallas_call(
        _tc_stage_d,
        grid=(nblk,),
        in_specs=[
            pl.BlockSpec((NC, 128, D2), lambda i: (0, i, 0)),
            pl.BlockSpec((1, NCLASS), lambda i: (0, 0)),
        ],
        out_specs=pl.BlockSpec((128, NCLASS), lambda i: (i, 0)),
        out_shape=jax.ShapeDtypeStruct((NPAD, NCLASS), _f32),
    )(part2, b2row)

    return out[:N]
